# Initial kernel scaffold; baseline (speedup 1.0000x reference)
import jax
import jax.numpy as jnp
from jax.experimental import pallas as pl


def kernel(x, memories):
    raise NotImplementedError("write your pallas kernel here")


# speedup vs baseline: 19.9673x; 19.9673x over previous
import functools

import jax
import jax.numpy as jnp
from jax import lax
from jax.experimental import pallas as pl
from jax.experimental.pallas import tpu as pltpu
from jax.experimental.pallas import tpu_sc as plsc

NUM_MEM = 100000
HID = 64
K = 32
CHUNK = 128
M_PAD = 100352
N_CHUNKS = M_PAD // CHUNK
QB = 8
NGROUP = 8
GW = M_PAD // NGROUP
GC = N_CHUNKS // NGROUP
NEG = -3.0
MASKED = -1e30

_HIGH = lax.Precision.HIGHEST


def _tc_body(x_ref, memt_hbm, attn_ref, topi_ref, topv_ref,
             memt_ref, log_ref, sem):
    step = pl.program_id(0)

    @pl.when(step == 0)
    def _init():
        cp = pltpu.make_async_copy(memt_hbm, memt_ref, sem)
        cp.start()
        cp.wait()

    xn = x_ref[...]

    cms = []
    for g in range(NGROUP):
        sl = pl.ds(g * GW, GW)
        raw = lax.dot_general(xn, memt_ref[:, sl], (((1,), (0,)), ((), ())))
        col = g * GW + lax.broadcasted_iota(jnp.int32, (QB, GW), 1)
        lg = jnp.where(col < NUM_MEM, raw, NEG)
        log_ref[:, sl] = lg
        cms.append(jnp.max(lg.reshape(QB, GC, CHUNK), axis=2))
    cm = jnp.concatenate(cms, axis=1)
    gmax = jnp.max(cm, axis=1, keepdims=True)

    iota_cm = lax.broadcasted_iota(
        jnp.int32, (QB, N_CHUNKS), 1).astype(jnp.float32)
    ohs = []
    cidxs = []
    cmw = cm
    for _ in range(K):
        v = jnp.max(cmw, axis=1, keepdims=True)
        hit = cmw == v
        first = jnp.min(jnp.where(hit, iota_cm, 1e9), axis=1, keepdims=True)
        hit1 = hit & (iota_cm == first)
        ohs.append(jnp.where(hit1, 1.0, 0.0)[:, None, :])
        cidxs.append(first)
        cmw = jnp.where(hit1, MASKED, cmw)
    oh = jnp.concatenate(ohs, axis=1)
    cidx = jnp.concatenate(cidxs, axis=1)

    cand = jnp.zeros((QB, K, CHUNK), jnp.float32)
    for g in range(NGROUP):
        ohg = lax.slice_in_dim(oh, g * GC, (g + 1) * GC, axis=2)
        lg3 = log_ref[:, pl.ds(g * GW, GW)].reshape(QB, GC, CHUNK)
        cand = cand + lax.dot_general(ohg, lg3, (((2,), (1,)), ((0,), (0,))),
                                      precision=_HIGH)
    candf = cand.reshape(QB, K * CHUNK)
    lane = jnp.bitwise_and(
        lax.broadcasted_iota(jnp.int32, (QB, K * CHUNK), 1),
        CHUNK - 1).astype(jnp.float32)
    base = jnp.broadcast_to(cidx[:, :, None],
                            (QB, K, CHUNK)).reshape(QB, K * CHUNK)
    colidf = base * float(CHUNK) + lane

    cw = candf
    vals = []
    ids = []
    sel = jnp.zeros((QB, K * CHUNK), jnp.float32)
    for _ in range(K):
        v = jnp.max(cw, axis=1, keepdims=True)
        hit = cw == v
        first = jnp.min(jnp.where(hit, colidf, 1e9), axis=1, keepdims=True)
        hit1 = hit & (colidf == first)
        ids.append(first)
        vals.append(v)
        sel = sel + jnp.where(hit1, 1.0, 0.0)
        cw = jnp.where(hit1, MASKED, cw)
    topval = jnp.concatenate(vals, axis=1)
    topcol = jnp.concatenate(ids, axis=1)

    ex = sel * jnp.exp(candf - gmax)
    s = jnp.sum(ex, axis=1, keepdims=True)
    candw = (ex / s).reshape(QB, K, CHUNK)

    topi_ref[...] = jnp.clip(topcol, 0.0, float(NUM_MEM - 1)).astype(jnp.int32)
    topv_ref[...] = jnp.exp(topval - gmax) / s

    for g in range(NGROUP):
        ohg = lax.slice_in_dim(oh, g * GC, (g + 1) * GC, axis=2)
        d3 = lax.dot_general(ohg, candw, (((1,), (1,)), ((0,), (0,))),
                             precision=_HIGH)
        d2 = d3.reshape(QB, GW)
        lo = g * GW
        w = min(GW, NUM_MEM - lo)
        attn_ref[:, lo:lo + w] = d2[:, :w]


_NC = 2
_NS = 16
_NW = _NC * _NS
_QW = 1024 // _NW
_E = _QW * K


def _sc_body(topi_hbm, topv_hbm, mem_hbm, read_hbm,
             idx_v, wgt_v, rows_v, read_v, sem):
    w = lax.axis_index("s") * _NC + lax.axis_index("c")
    pltpu.sync_copy(topi_hbm.at[w], idx_v)
    pltpu.sync_copy(topv_hbm.at[w], wgt_v)
    cp = pltpu.make_async_copy(mem_hbm.at[idx_v], rows_v, sem)
    cp.start()
    cp.wait()

    iota16 = lax.iota(jnp.int32, 16)

    def qloop(q, _):
        base = q * K
        w16 = [wgt_v[pl.ds(base + h * 16, 16)] for h in range(K // 16)]
        accs = [jnp.zeros((16,), jnp.float32) for _ in range(4)]
        for k in range(K):
            wv = lax.gather(
                w16[k // 16], jnp.full((16, 1), k % 16, jnp.int32),
                lax.GatherDimensionNumbers(offset_dims=(),
                                           collapsed_slice_dims=(0,),
                                           start_index_map=(0,)),
                (1,), mode=lax.GatherScatterMode.PROMISE_IN_BOUNDS)
            for c in range(4):
                accs[c] = accs[c] + wv * rows_v[base + k, pl.ds(c * 16, 16)]
        for c in range(4):
            read_v[q, pl.ds(c * 16, 16)] = accs[c]
        return 0

    lax.fori_loop(0, _QW, qloop, 0)
    pltpu.sync_copy(read_v, read_hbm.at[pl.ds(w * _QW, _QW)])


@jax.jit
def kernel(x, memories):
    B = x.shape[0]
    x_norm = x / jnp.clip(jnp.linalg.norm(x, axis=-1, keepdims=True), 1e-12)
    m_norm = memories / jnp.clip(
        jnp.linalg.norm(memories, axis=-1, keepdims=True), 1e-12)
    xq = x_norm.reshape(B, HID)
    memt = jnp.pad(m_norm, ((0, M_PAD - NUM_MEM), (0, 0))).T

    grid = B // QB
    attn, topi, topv = pl.pallas_call(
        _tc_body,
        grid=(grid,),
        in_specs=[
            pl.BlockSpec((QB, HID), lambda i: (i, 0)),
            pl.BlockSpec(memory_space=pl.ANY),
        ],
        out_specs=[
            pl.BlockSpec((QB, NUM_MEM), lambda i: (i, 0)),
            pl.BlockSpec((QB, K), lambda i: (i, 0)),
            pl.BlockSpec((QB, K), lambda i: (i, 0)),
        ],
        out_shape=[
            jax.ShapeDtypeStruct((B, NUM_MEM), jnp.float32),
            jax.ShapeDtypeStruct((B, K), jnp.int32),
            jax.ShapeDtypeStruct((B, K), jnp.float32),
        ],
        scratch_shapes=[
            pltpu.VMEM((HID, M_PAD), jnp.float32),
            pltpu.VMEM((QB, M_PAD), jnp.float32),
            pltpu.SemaphoreType.DMA,
        ],
    )(xq, memt)

    mesh = plsc.VectorSubcoreMesh(core_axis_name="c", subcore_axis_name="s")
    read = pl.kernel(
        _sc_body,
        out_type=jax.ShapeDtypeStruct((B, HID), jnp.float32),
        mesh=mesh,
        compiler_params=pltpu.CompilerParams(use_tc_tiling_on_sc=False),
        scratch_types=[
            pltpu.VMEM((_E,), jnp.int32),
            pltpu.VMEM((_E,), jnp.float32),
            pltpu.VMEM((_E, HID), jnp.float32),
            pltpu.VMEM((_QW, HID), jnp.float32),
            pltpu.SemaphoreType.DMA,
        ],
    )(topi.reshape(_NW, _E), topv.reshape(_NW, _E), memories)

    return (read.reshape(B, 1, HID), attn.reshape(B, 1, NUM_MEM))


# speedup vs baseline: 25.7189x; 1.2880x over previous
import functools

import jax
import jax.numpy as jnp
from jax import lax
from jax.experimental import pallas as pl
from jax.experimental.pallas import tpu as pltpu
from jax.experimental.pallas import tpu_sc as plsc

NUM_MEM = 100000
HID = 64
K = 32
CHUNK = 128
M_PAD = 100352
N_CHUNKS = M_PAD // CHUNK
QB = 16
NGROUP = 4
GW = M_PAD // NGROUP
GC = N_CHUNKS // NGROUP
NEG = -3.0
MASKED = -1e30

_HIGH = lax.Precision.HIGHEST


def _tc_body(x_ref, memt_hbm, attn_ref, topi_ref, topv_ref,
             memt_ref, log_ref, sem):
    step = pl.program_id(0)

    @pl.when(step == 0)
    def _init():
        cp = pltpu.make_async_copy(memt_hbm, memt_ref, sem)
        cp.start()
        cp.wait()

    xn = x_ref[...]

    cms = []
    for g in range(NGROUP):
        sl = pl.ds(g * GW, GW)
        raw = lax.dot_general(xn, memt_ref[:, sl], (((1,), (0,)), ((), ())))
        col = g * GW + lax.broadcasted_iota(jnp.int32, (QB, GW), 1)
        lg = jnp.where(col < NUM_MEM, raw, NEG)
        log_ref[:, sl] = lg
        cms.append(jnp.max(lg.reshape(QB, GC, CHUNK), axis=2))
    cm = jnp.concatenate(cms, axis=1)
    gmax = jnp.max(cm, axis=1, keepdims=True)

    iota_cm = lax.broadcasted_iota(
        jnp.int32, (QB, N_CHUNKS), 1).astype(jnp.float32)
    ohs = []
    cidxs = []
    cmw = cm
    for _ in range(K):
        v = jnp.max(cmw, axis=1, keepdims=True)
        hit = cmw == v
        first = jnp.min(jnp.where(hit, iota_cm, 1e9), axis=1, keepdims=True)
        hit1 = hit & (iota_cm == first)
        ohs.append(jnp.where(hit1, 1.0, 0.0)[:, None, :])
        cidxs.append(first)
        cmw = jnp.where(hit1, MASKED, cmw)
    oh = jnp.concatenate(ohs, axis=1)
    cidx = jnp.concatenate(cidxs, axis=1)

    cand = jnp.zeros((QB, K, CHUNK), jnp.float32)
    for g in range(NGROUP):
        ohg = lax.slice_in_dim(oh, g * GC, (g + 1) * GC, axis=2)
        lg3 = log_ref[:, pl.ds(g * GW, GW)].reshape(QB, GC, CHUNK)
        cand = cand + lax.dot_general(ohg, lg3, (((2,), (1,)), ((0,), (0,))),
                                      precision=_HIGH)
    candf = cand.reshape(QB, K * CHUNK)
    lane = jnp.bitwise_and(
        lax.broadcasted_iota(jnp.int32, (QB, K * CHUNK), 1),
        CHUNK - 1).astype(jnp.float32)
    base = jnp.broadcast_to(cidx[:, :, None],
                            (QB, K, CHUNK)).reshape(QB, K * CHUNK)
    colidf = base * float(CHUNK) + lane

    cw = candf
    vals = []
    ids = []
    sel = jnp.zeros((QB, K * CHUNK), jnp.float32)
    for _ in range(K):
        v = jnp.max(cw, axis=1, keepdims=True)
        hit = cw == v
        first = jnp.min(jnp.where(hit, colidf, 1e9), axis=1, keepdims=True)
        hit1 = hit & (colidf == first)
        ids.append(first)
        vals.append(v)
        sel = sel + jnp.where(hit1, 1.0, 0.0)
        cw = jnp.where(hit1, MASKED, cw)
    topval = jnp.concatenate(vals, axis=1)
    topcol = jnp.concatenate(ids, axis=1)

    ex = sel * jnp.exp(candf - gmax)
    s = jnp.sum(ex, axis=1, keepdims=True)
    candw = (ex / s).reshape(QB, K, CHUNK)

    topi_ref[...] = jnp.clip(topcol, 0.0, float(NUM_MEM - 1)).astype(jnp.int32)
    topv_ref[...] = jnp.exp(topval - gmax) / s

    for g in range(NGROUP):
        ohg = lax.slice_in_dim(oh, g * GC, (g + 1) * GC, axis=2)
        d3 = lax.dot_general(ohg, candw, (((1,), (1,)), ((0,), (0,))),
                             precision=_HIGH)
        d2 = d3.reshape(QB, GW)
        lo = g * GW
        w = min(GW, NUM_MEM - lo)
        attn_ref[:, lo:lo + w] = d2[:, :w]


_NC = 2
_NS = 16
_NW = _NC * _NS
_QW = 1024 // _NW
_E = _QW * K


def _sc_body(topi_hbm, topv_hbm, mem_hbm, read_hbm,
             idx_v, wgt_v, rows_v, read_v, sem):
    w = lax.axis_index("s") * _NC + lax.axis_index("c")
    pltpu.sync_copy(topi_hbm.at[w], idx_v)
    pltpu.sync_copy(topv_hbm.at[w], wgt_v)
    cp = pltpu.make_async_copy(mem_hbm.at[idx_v], rows_v, sem)
    cp.start()
    cp.wait()

    iota16 = lax.iota(jnp.int32, 16)

    def qloop(q, _):
        base = q * K
        w16 = [wgt_v[pl.ds(base + h * 16, 16)] for h in range(K // 16)]
        accs = [jnp.zeros((16,), jnp.float32) for _ in range(4)]
        for k in range(K):
            wv = lax.gather(
                w16[k // 16], jnp.full((16, 1), k % 16, jnp.int32),
                lax.GatherDimensionNumbers(offset_dims=(),
                                           collapsed_slice_dims=(0,),
                                           start_index_map=(0,)),
                (1,), mode=lax.GatherScatterMode.PROMISE_IN_BOUNDS)
            for c in range(4):
                accs[c] = accs[c] + wv * rows_v[base + k, pl.ds(c * 16, 16)]
        for c in range(4):
            read_v[q, pl.ds(c * 16, 16)] = accs[c]
        return 0

    lax.fori_loop(0, _QW, qloop, 0)
    pltpu.sync_copy(read_v, read_hbm.at[pl.ds(w * _QW, _QW)])


@jax.jit
def kernel(x, memories):
    B = x.shape[0]
    x_norm = x / jnp.clip(jnp.linalg.norm(x, axis=-1, keepdims=True), 1e-12)
    m_norm = memories / jnp.clip(
        jnp.linalg.norm(memories, axis=-1, keepdims=True), 1e-12)
    xq = x_norm.reshape(B, HID)
    memt = jnp.pad(m_norm, ((0, M_PAD - NUM_MEM), (0, 0))).T

    grid = B // QB
    attn, topi, topv = pl.pallas_call(
        _tc_body,
        grid=(grid,),
        in_specs=[
            pl.BlockSpec((QB, HID), lambda i: (i, 0)),
            pl.BlockSpec(memory_space=pl.ANY),
        ],
        out_specs=[
            pl.BlockSpec((QB, NUM_MEM), lambda i: (i, 0)),
            pl.BlockSpec((QB, K), lambda i: (i, 0)),
            pl.BlockSpec((QB, K), lambda i: (i, 0)),
        ],
        out_shape=[
            jax.ShapeDtypeStruct((B, NUM_MEM), jnp.float32),
            jax.ShapeDtypeStruct((B, K), jnp.int32),
            jax.ShapeDtypeStruct((B, K), jnp.float32),
        ],
        scratch_shapes=[
            pltpu.VMEM((HID, M_PAD), jnp.float32),
            pltpu.VMEM((QB, M_PAD), jnp.float32),
            pltpu.SemaphoreType.DMA,
        ],
    )(xq, memt)

    mesh = plsc.VectorSubcoreMesh(core_axis_name="c", subcore_axis_name="s")
    read = pl.kernel(
        _sc_body,
        out_type=jax.ShapeDtypeStruct((B, HID), jnp.float32),
        mesh=mesh,
        compiler_params=pltpu.CompilerParams(use_tc_tiling_on_sc=False),
        scratch_types=[
            pltpu.VMEM((_E,), jnp.int32),
            pltpu.VMEM((_E,), jnp.float32),
            pltpu.VMEM((_E, HID), jnp.float32),
            pltpu.VMEM((_QW, HID), jnp.float32),
            pltpu.SemaphoreType.DMA,
        ],
    )(topi.reshape(_NW, _E), topv.reshape(_NW, _E), memories)

    return (read.reshape(B, 1, HID), attn.reshape(B, 1, NUM_MEM))


# speedup vs baseline: 36.8084x; 1.4312x over previous
import functools

import jax
import jax.numpy as jnp
from jax import lax
from jax.experimental import pallas as pl
from jax.experimental.pallas import tpu as pltpu
from jax.experimental.pallas import tpu_sc as plsc

NUM_MEM = 100000
HID = 64
K = 32
CHUNK = 128
M_PAD = 100352
N_CHUNKS = M_PAD // CHUNK
QB = 16
NGROUP = 2
GW = M_PAD // NGROUP
GC = N_CHUNKS // NGROUP
NEG = -3.0
MASKED = -1e30

_HIGH = lax.Precision.HIGHEST


def _tc_body(x_ref, memt_hbm, attn_ref, topi_ref, topv_ref,
             memt_ref, log_ref, sem):
    step = pl.program_id(0)

    @pl.when(step == 0)
    def _init():
        cp = pltpu.make_async_copy(memt_hbm, memt_ref, sem)
        cp.start()
        cp.wait()

    xn = x_ref[...]

    cms = []
    for g in range(NGROUP):
        sl = pl.ds(g * GW, GW)
        raw = lax.dot_general(xn, memt_ref[:, sl], (((1,), (0,)), ((), ())))
        col = g * GW + lax.broadcasted_iota(jnp.int32, (QB, GW), 1)
        lg = jnp.where(col < NUM_MEM, raw, NEG)
        log_ref[:, sl] = lg
        cms.append(jnp.max(lg.reshape(QB, GC, CHUNK), axis=2))
    cm = jnp.concatenate(cms, axis=1)
    gmax = jnp.max(cm, axis=1, keepdims=True)

    iota_cm = lax.broadcasted_iota(
        jnp.int32, (QB, N_CHUNKS), 1).astype(jnp.float32)
    ohs = []
    cidxs = []
    cmw = cm
    for _ in range(K):
        v = jnp.max(cmw, axis=1, keepdims=True)
        hit = cmw == v
        first = jnp.min(jnp.where(hit, iota_cm, 1e9), axis=1, keepdims=True)
        hit1 = hit & (iota_cm == first)
        ohs.append(jnp.where(hit1, 1.0, 0.0)[:, None, :])
        cidxs.append(first)
        cmw = jnp.where(hit1, MASKED, cmw)
    oh = jnp.concatenate(ohs, axis=1)
    cidx = jnp.concatenate(cidxs, axis=1)

    cand = jnp.zeros((QB, K, CHUNK), jnp.float32)
    for g in range(NGROUP):
        ohg = lax.slice_in_dim(oh, g * GC, (g + 1) * GC, axis=2)
        lg3 = log_ref[:, pl.ds(g * GW, GW)].reshape(QB, GC, CHUNK)
        cand = cand + lax.dot_general(ohg, lg3, (((2,), (1,)), ((0,), (0,))),
                                      precision=_HIGH)
    candf = cand.reshape(QB, K * CHUNK)
    lane = jnp.bitwise_and(
        lax.broadcasted_iota(jnp.int32, (QB, K * CHUNK), 1),
        CHUNK - 1).astype(jnp.float32)
    base = jnp.broadcast_to(cidx[:, :, None],
                            (QB, K, CHUNK)).reshape(QB, K * CHUNK)
    colidf = base * float(CHUNK) + lane

    cw = candf
    vals = []
    ids = []
    sel = jnp.zeros((QB, K * CHUNK), jnp.float32)
    for _ in range(K):
        v = jnp.max(cw, axis=1, keepdims=True)
        hit = cw == v
        first = jnp.min(jnp.where(hit, colidf, 1e9), axis=1, keepdims=True)
        hit1 = hit & (colidf == first)
        ids.append(first)
        vals.append(v)
        sel = sel + jnp.where(hit1, 1.0, 0.0)
        cw = jnp.where(hit1, MASKED, cw)
    topval = jnp.concatenate(vals, axis=1)
    topcol = jnp.concatenate(ids, axis=1)

    ex = sel * jnp.exp(candf - gmax)
    s = jnp.sum(ex, axis=1, keepdims=True)
    candw = (ex / s).reshape(QB, K, CHUNK)

    topi_ref[...] = jnp.clip(topcol, 0.0, float(NUM_MEM - 1)).astype(jnp.int32)
    topv_ref[...] = jnp.exp(topval - gmax) / s

    for g in range(NGROUP):
        ohg = lax.slice_in_dim(oh, g * GC, (g + 1) * GC, axis=2)
        d3 = lax.dot_general(ohg, candw,
                             (((1,), (1,)), ((0,), (0,))))
        d2 = d3.reshape(QB, GW)
        lo = g * GW
        w = min(GW, NUM_MEM - lo)
        attn_ref[:, lo:lo + w] = d2[:, :w]


_NC = 2
_NS = 16
_NW = _NC * _NS
_QW = 1024 // _NW
_E = _QW * K


def _sc_body(topi_hbm, topv_hbm, mem_hbm, read_hbm,
             idx_v, wgt_v, rows_v, read_v, sem):
    w = lax.axis_index("s") * _NC + lax.axis_index("c")
    pltpu.sync_copy(topi_hbm.at[w], idx_v)
    pltpu.sync_copy(topv_hbm.at[w], wgt_v)
    cp = pltpu.make_async_copy(mem_hbm.at[idx_v], rows_v, sem)
    cp.start()
    cp.wait()

    iota16 = lax.iota(jnp.int32, 16)

    def qloop(q, _):
        base = q * K
        w16 = [wgt_v[pl.ds(base + h * 16, 16)] for h in range(K // 16)]
        accs = [jnp.zeros((16,), jnp.float32) for _ in range(4)]
        for k in range(K):
            wv = lax.gather(
                w16[k // 16], jnp.full((16, 1), k % 16, jnp.int32),
                lax.GatherDimensionNumbers(offset_dims=(),
                                           collapsed_slice_dims=(0,),
                                           start_index_map=(0,)),
                (1,), mode=lax.GatherScatterMode.PROMISE_IN_BOUNDS)
            for c in range(4):
                accs[c] = accs[c] + wv * rows_v[base + k, pl.ds(c * 16, 16)]
        for c in range(4):
            read_v[q, pl.ds(c * 16, 16)] = accs[c]
        return 0

    lax.fori_loop(0, _QW, qloop, 0)
    pltpu.sync_copy(read_v, read_hbm.at[pl.ds(w * _QW, _QW)])


@jax.jit
def kernel(x, memories):
    B = x.shape[0]
    x_norm = x / jnp.clip(jnp.linalg.norm(x, axis=-1, keepdims=True), 1e-12)
    m_norm = memories / jnp.clip(
        jnp.linalg.norm(memories, axis=-1, keepdims=True), 1e-12)
    xq = x_norm.reshape(B, HID)
    memt = jnp.pad(m_norm, ((0, M_PAD - NUM_MEM), (0, 0))).T

    grid = B // QB
    attn, topi, topv = pl.pallas_call(
        _tc_body,
        grid=(grid,),
        in_specs=[
            pl.BlockSpec((QB, HID), lambda i: (i, 0)),
            pl.BlockSpec(memory_space=pl.ANY),
        ],
        out_specs=[
            pl.BlockSpec((QB, NUM_MEM), lambda i: (i, 0)),
            pl.BlockSpec((QB, K), lambda i: (i, 0)),
            pl.BlockSpec((QB, K), lambda i: (i, 0)),
        ],
        out_shape=[
            jax.ShapeDtypeStruct((B, NUM_MEM), jnp.float32),
            jax.ShapeDtypeStruct((B, K), jnp.int32),
            jax.ShapeDtypeStruct((B, K), jnp.float32),
        ],
        scratch_shapes=[
            pltpu.VMEM((HID, M_PAD), jnp.float32),
            pltpu.VMEM((QB, M_PAD), jnp.float32),
            pltpu.SemaphoreType.DMA,
        ],
    )(xq, memt)

    mesh = plsc.VectorSubcoreMesh(core_axis_name="c", subcore_axis_name="s")
    read = pl.kernel(
        _sc_body,
        out_type=jax.ShapeDtypeStruct((B, HID), jnp.float32),
        mesh=mesh,
        compiler_params=pltpu.CompilerParams(use_tc_tiling_on_sc=False),
        scratch_types=[
            pltpu.VMEM((_E,), jnp.int32),
            pltpu.VMEM((_E,), jnp.float32),
            pltpu.VMEM((_E, HID), jnp.float32),
            pltpu.VMEM((_QW, HID), jnp.float32),
            pltpu.SemaphoreType.DMA,
        ],
    )(topi.reshape(_NW, _E), topv.reshape(_NW, _E), memories)

    return (read.reshape(B, 1, HID), attn.reshape(B, 1, NUM_MEM))
